# baseline (speedup 1.0000x reference)
import functools

import numpy as np
import jax
import jax.numpy as jnp
from jax import lax
from jax.experimental import pallas as pl
from jax.experimental.pallas import tpu as pltpu
from jax.experimental.pallas import tpu_sc as plsc

N = 10000
D_IN = 256
H = 64
ATTN = 64
L = 4
E = 160000
MT = 2

NC = 2
NS = 16
CHUNK = 128
CPS = 80
E_PAD = NS * CPS * CHUNK
N_PAD = 10240
ZR = N_PAD // NS


@functools.cache
def _get_sc_counts():
    mesh = plsc.VectorSubcoreMesh(core_axis_name="c", subcore_axis_name="s")
    return pl.kernel(
        _sc_counts_body,
        out_type=jax.ShapeDtypeStruct((MT, N_PAD, 16), jnp.float32),
        mesh=mesh,
        compiler_params=pltpu.CompilerParams(use_tc_tiling_on_sc=False,
                                             needs_layout_passes=False),
        scratch_types=[
            pltpu.VMEM((CPS, CHUNK), jnp.int32),
            pltpu.VMEM((CHUNK, 16), jnp.float32),
            pltpu.VMEM_SHARED((N_PAD, 16), jnp.float32),
        ],
    )


def _sc_counts_body(dst0, dst1, z16, ones16, cnt_out, dstv, onesv, acc):
    c = lax.axis_index("c")
    s = lax.axis_index("s")
    pltpu.sync_copy(z16, acc.at[pl.ds(s * ZR, ZR)])
    pltpu.sync_copy(ones16, onesv)

    @pl.when(c == 0)
    def _():
        pltpu.sync_copy(dst0.at[pl.ds(s * CPS, CPS)], dstv)

    @pl.when(c == 1)
    def _():
        pltpu.sync_copy(dst1.at[pl.ds(s * CPS, CPS)], dstv)

    plsc.subcore_barrier()

    def step(i, carry):
        pltpu.sync_copy(onesv, acc.at[dstv.at[i]], add=True)
        return carry

    lax.fori_loop(0, CPS, step, 0)
    plsc.subcore_barrier()
    pltpu.sync_copy(acc.at[pl.ds(s * ZR, ZR)], cnt_out.at[c, pl.ds(s * ZR, ZR)])


@functools.cache
def _get_sc_agg():
    mesh = plsc.VectorSubcoreMesh(core_axis_name="c", subcore_axis_name="s")
    return pl.kernel(
        _sc_agg_body,
        out_type=jax.ShapeDtypeStruct((MT, N_PAD, H), jnp.float32),
        mesh=mesh,
        compiler_params=pltpu.CompilerParams(use_tc_tiling_on_sc=False,
                                             needs_layout_passes=False),
        scratch_types=[
            pltpu.VMEM((CPS, CHUNK), jnp.int32),
            pltpu.VMEM((CPS, CHUNK), jnp.int32),
            *([pltpu.VMEM((CHUNK, H), jnp.bfloat16)] * 4),
            *([pltpu.VMEM((CHUNK, H), jnp.float32)] * 4),
            pltpu.VMEM_SHARED((N_PAD, H), jnp.float32),
            *([pltpu.SemaphoreType.DMA] * 8),
        ],
    )


NBUF = 4


def _sc_agg_body(p0, p1, src0, dst0, src1, dst1, z64, agg_out,
                 srcv, dstv, r0, r1, r2, r3, t0, t1, t2, t3, acc,
                 s0_, s1_, s2_, s3_, u0, u1, u2, u3):
    c = lax.axis_index("c")
    s = lax.axis_index("s")
    pltpu.sync_copy(z64, acc.at[pl.ds(s * ZR, ZR)])

    @pl.when(c == 0)
    def _():
        pltpu.sync_copy(src0.at[pl.ds(s * CPS, CPS)], srcv)
        pltpu.sync_copy(dst0.at[pl.ds(s * CPS, CPS)], dstv)

    @pl.when(c == 1)
    def _():
        pltpu.sync_copy(src1.at[pl.ds(s * CPS, CPS)], srcv)
        pltpu.sync_copy(dst1.at[pl.ds(s * CPS, CPS)], dstv)

    plsc.subcore_barrier()

    bufs = ((r0, s0_), (r1, s1_), (r2, s2_), (r3, s3_))
    stages = ((t0, u0), (t1, u1), (t2, u2), (t3, u3))

    def run(p):
        def process(prev, b):
            buf, gsem = bufs[b]
            stg, ssem = stages[b]
            pltpu.make_async_copy(p.at[srcv.at[prev]], buf, gsem).wait()
            pltpu.make_async_copy(stg, acc.at[dstv.at[prev]], ssem).wait()

            @plsc.parallel_loop(0, CHUNK, unroll=4)
            def _(r):
                for g in range(2):
                    v = buf[r, pl.ds(g * 32, 32)]
                    ev, od = plsc.unpack(v, format=plsc.PackFormat.INTERLEAVED)
                    stg[r, pl.ds(g * 32, 16)] = ev
                    stg[r, pl.ds(g * 32 + 16, 16)] = od

            pltpu.async_copy(stg, acc.at[dstv.at[prev]], ssem)

        for b in range(NBUF):
            buf, gsem = bufs[b]
            stg, ssem = stages[b]
            pltpu.sync_copy(z64.at[pl.ds(0, CHUNK)], stg)
            pltpu.async_copy(stg, acc.at[dstv.at[b]], ssem)
            pltpu.async_copy(p.at[srcv.at[b]], buf, gsem)

        def group(j, carry):
            for b in range(NBUF):
                process(NBUF * (j - 1) + b, b)
                pltpu.async_copy(p.at[srcv.at[NBUF * j + b]],
                                 bufs[b][0], bufs[b][1])
            return carry

        lax.fori_loop(1, CPS // NBUF, group, 0)
        for b in range(NBUF):
            process(CPS - NBUF + b, b)
        for b in range(NBUF):
            stg, ssem = stages[b]
            pltpu.make_async_copy(
                stg, acc.at[dstv.at[CPS - NBUF + b]], ssem).wait()

    @pl.when(c == 0)
    def _():
        run(p0)

    @pl.when(c == 1)
    def _():
        run(p1)

    plsc.subcore_barrier()
    pltpu.sync_copy(acc.at[pl.ds(s * ZR, ZR)], agg_out.at[c, pl.ds(s * ZR, ZR)])


def _dot_nt(a, b):
    return lax.dot_general(a, b, (((1,), (1,)), ((), ())),
                           preferred_element_type=jnp.float32)


def _dot_nn(a, b):
    return lax.dot_general(a, b, (((1,), (0,)), ((), ())),
                           preferred_element_type=jnp.float32)


def _tc_pre_body(x_ref, wsrc_ref, wu_ref, p0_ref, p1_ref):
    x = x_ref[...]
    outs = [p0_ref, p1_ref]
    for m in range(MT):
        wu2 = wu_ref[m, :, H:]
        b_mat = _dot_nn(wu2, wsrc_ref[m])
        outs[m][...] = _dot_nt(x, b_mat).astype(jnp.bfloat16)


def _tc_prep_body(wdst0_ref, wdstr_ref, wu_ref, wun_ref, wsrcr_ref,
                  bdst_ref, bsrc_ref, bu_ref,
                  a0_ref, ar_ref, br_ref, cv_ref):
    for i in range(L):
        for m in range(MT):
            wu1 = wu_ref[i, m, :, :H]
            wu2 = wu_ref[i, m, :, H:]
            if i == 0:
                a0_ref[m] = _dot_nn(wu1, wdst0_ref[m])
            else:
                ar_ref[i - 1, m] = _dot_nn(wu1, wdstr_ref[i - 1, m])
                br_ref[i - 1, m] = _dot_nn(wun_ref[i - 1, m, :, H:],
                                           wsrcr_ref[i - 1, m])
            cv_ref[i, m] = (bu_ref[i, m] + _dot_nt(bdst_ref[i, m], wu1)
                            + _dot_nt(bsrc_ref[i, m], wu2))


def _tc_layer_body(is_last, din,
                   h_ref, agg_ref, cnt_ref,
                   a_ref, cv_ref,
                   wa1_ref, ba1_ref, wa2_ref, gamma_ref, beta_ref,
                   *rest):
    if is_last:
        wfc_ref, bfc_ref, out_ref = rest
    else:
        bn_ref, h_out_ref, p0_ref, p1_ref = rest

    h = h_ref[...]
    embs = []
    scores = []
    for m in range(MT):
        inv = 1.0 / jnp.maximum(cnt_ref[m, :N, 0:1], 1.0)
        emb = (_dot_nt(h, a_ref[m]) + agg_ref[m, :N, :] * inv
               + cv_ref[m])
        t = jnp.tanh(_dot_nt(emb, wa1_ref[...]) + ba1_ref[...])
        scores.append(jnp.sum(t * wa2_ref[...]) * (1.0 / N))
        embs.append(emb)

    a0 = 1.0 / (1.0 + jnp.exp(scores[1] - scores[0]))
    hc = a0 * embs[0] + (1.0 - a0) * embs[1]

    mu = jnp.sum(hc, axis=0, keepdims=True) * (1.0 / N)
    d = hc - mu
    var = jnp.sum(d * d, axis=0, keepdims=True) * (1.0 / N)
    hn = d * lax.rsqrt(var + 1.0) * gamma_ref[...] + beta_ref[...]
    ho = jnp.where(hn >= 0, hn, 0.01 * hn)

    if is_last:
        out_ref[...] = (jnp.sum(ho * wfc_ref[...], axis=1, keepdims=True)
                        + bfc_ref[...])
    else:
        h_out_ref[...] = ho
        outs = [p0_ref, p1_ref]
        for m in range(MT):
            outs[m][...] = _dot_nt(ho, bn_ref[m]).astype(jnp.bfloat16)


_TC_PARAMS = pltpu.CompilerParams(vmem_limit_bytes=100 * 1024 * 1024)

_tc_pre = pl.pallas_call(
    _tc_pre_body,
    out_shape=[jax.ShapeDtypeStruct((N, H), jnp.bfloat16)] * MT,
    compiler_params=_TC_PARAMS,
)

_tc_prep = pl.pallas_call(
    _tc_prep_body,
    out_shape=[
        jax.ShapeDtypeStruct((MT, H, D_IN), jnp.float32),
        jax.ShapeDtypeStruct((L - 1, MT, H, H), jnp.float32),
        jax.ShapeDtypeStruct((L - 1, MT, H, H), jnp.float32),
        jax.ShapeDtypeStruct((L, MT, 1, H), jnp.float32),
    ],
    compiler_params=_TC_PARAMS,
)


def _make_tc_layer(is_last, din):
    if is_last:
        out_shape = jax.ShapeDtypeStruct((N, 1), jnp.float32)
    else:
        out_shape = [jax.ShapeDtypeStruct((N, H), jnp.float32),
                     jax.ShapeDtypeStruct((N, H), jnp.bfloat16),
                     jax.ShapeDtypeStruct((N, H), jnp.bfloat16)]
    return pl.pallas_call(
        functools.partial(_tc_layer_body, is_last, din),
        out_shape=out_shape,
        compiler_params=_TC_PARAMS,
    )


_tc_layer_first = _make_tc_layer(False, D_IN)
_tc_layer_mid = _make_tc_layer(False, H)
_tc_layer_last = _make_tc_layer(True, H)


def _stack(layer, name, key):
    return jnp.stack([layer["convs"][m][name][key] for m in range(MT)])


def _stack_bias(layer, name):
    return jnp.stack([layer["convs"][m][name]["b"][None, :] for m in range(MT)])


_PERM = np.concatenate(
    [np.concatenate([32 * g + 2 * np.arange(16), 32 * g + 2 * np.arange(16) + 1])
     for g in range(H // 32)])


def kernel(x, edge_index_0, edge_index_1, params):
    pad = E_PAD - E
    srcs, dsts = [], []
    dummy = N + (jnp.arange(pad, dtype=jnp.int32) % (N_PAD - N))
    for ei in (edge_index_0, edge_index_1):
        src = jnp.concatenate([ei[0], jnp.zeros((pad,), jnp.int32)])
        dst = jnp.concatenate([ei[1], dummy])
        srcs.append(src.reshape(E_PAD // CHUNK, CHUNK))
        dsts.append(dst.reshape(E_PAD // CHUNK, CHUNK))

    z16 = jnp.zeros((ZR, 16), jnp.float32)
    ones16 = jnp.ones((CHUNK, 16), jnp.float32)
    z64 = jnp.zeros((ZR, H), jnp.float32)

    cnt = _get_sc_counts()(dsts[0], dsts[1], z16, ones16)

    layers = params["layers"]
    p0, p1 = _tc_pre(x, _stack(layers[0], "lin_src", "W"),
                     _stack(layers[0], "lin_update", "W"))

    a0, ar, br, cv = _tc_prep(
        _stack(layers[0], "lin_dst", "W"),
        jnp.stack([_stack(layers[i], "lin_dst", "W")[:, :, _PERM]
                   for i in range(1, L)]),
        jnp.stack([_stack(layers[i], "lin_update", "W")[:, _PERM, :]
                   for i in range(L)]),
        jnp.stack([_stack(layers[i], "lin_update", "W")
                   for i in range(1, L)]),
        jnp.stack([_stack(layers[i], "lin_src", "W")[:, :, _PERM]
                   for i in range(1, L)]),
        jnp.stack([_stack_bias(layers[i], "lin_dst") for i in range(L)]),
        jnp.stack([_stack_bias(layers[i], "lin_src") for i in range(L)]),
        jnp.stack([_stack_bias(layers[i], "lin_update")[:, :, _PERM]
                   for i in range(L)]),
    )

    h = x
    for i in range(L):
        lp = layers[i]
        agg = _get_sc_agg()(p0, p1, srcs[0], dsts[0], srcs[1], dsts[1], z64)
        common = (
            h, agg, cnt,
            a0 if i == 0 else ar[i - 1], cv[i],
            lp["attn1"]["W"][:, _PERM], lp["attn1"]["b"][None, :],
            lp["attn2"]["W"],
            lp["bn_gamma"][None, _PERM], lp["bn_beta"][None, _PERM],
        )
        if i == L - 1:
            out = _tc_layer_last(*common, params["fc"]["W"][:, _PERM],
                                 params["fc"]["b"][None, :])
        else:
            tc = _tc_layer_first if i == 0 else _tc_layer_mid
            h, p0, p1 = tc(*common, br[i])
    return out


# speedup vs baseline: 1.0903x; 1.0903x over previous
import functools

import numpy as np
import jax
import jax.numpy as jnp
from jax import lax
from jax.experimental import pallas as pl
from jax.experimental.pallas import tpu as pltpu
from jax.experimental.pallas import tpu_sc as plsc

N = 10000
D_IN = 256
H = 64
ATTN = 64
L = 4
E = 160000
MT = 2

NC = 2
NS = 16
CHUNK = 128
CPS = 80
E_PAD = NS * CPS * CHUNK
N_PAD = 10240
ZR = N_PAD // NS


@functools.cache
def _get_sc_counts():
    mesh = plsc.VectorSubcoreMesh(core_axis_name="c", subcore_axis_name="s")
    return pl.kernel(
        _sc_counts_body,
        out_type=jax.ShapeDtypeStruct((MT, N_PAD, 16), jnp.float32),
        mesh=mesh,
        compiler_params=pltpu.CompilerParams(use_tc_tiling_on_sc=False,
                                             needs_layout_passes=False),
        scratch_types=[
            pltpu.VMEM((CPS, CHUNK), jnp.int32),
            pltpu.VMEM((CHUNK, 16), jnp.float32),
            pltpu.VMEM_SHARED((N_PAD, 16), jnp.float32),
        ],
    )


def _sc_counts_body(dst0, dst1, z16, ones16, cnt_out, dstv, onesv, acc):
    c = lax.axis_index("c")
    s = lax.axis_index("s")
    pltpu.sync_copy(z16, acc.at[pl.ds(s * ZR, ZR)])
    pltpu.sync_copy(ones16, onesv)

    @pl.when(c == 0)
    def _():
        pltpu.sync_copy(dst0.at[pl.ds(s * CPS, CPS)], dstv)

    @pl.when(c == 1)
    def _():
        pltpu.sync_copy(dst1.at[pl.ds(s * CPS, CPS)], dstv)

    plsc.subcore_barrier()

    def step(i, carry):
        pltpu.sync_copy(onesv, acc.at[dstv.at[i]], add=True)
        return carry

    lax.fori_loop(0, CPS, step, 0)
    plsc.subcore_barrier()
    pltpu.sync_copy(acc.at[pl.ds(s * ZR, ZR)], cnt_out.at[c, pl.ds(s * ZR, ZR)])


@functools.cache
def _get_sc_agg():
    mesh = plsc.VectorSubcoreMesh(core_axis_name="c", subcore_axis_name="s")
    return pl.kernel(
        _sc_agg_body,
        out_type=jax.ShapeDtypeStruct((MT, N_PAD, H), jnp.float32),
        mesh=mesh,
        compiler_params=pltpu.CompilerParams(use_tc_tiling_on_sc=False,
                                             needs_layout_passes=False),
        scratch_types=[
            pltpu.VMEM((CPS, CHUNK), jnp.int32),
            pltpu.VMEM((CPS, CHUNK), jnp.int32),
            *([pltpu.VMEM((CHUNK, H), jnp.bfloat16)] * 4),
            *([pltpu.VMEM((CHUNK, H), jnp.float32)] * 4),
            pltpu.VMEM_SHARED((N_PAD, H), jnp.float32),
            *([pltpu.SemaphoreType.DMA] * 8),
        ],
    )


NBUF = 4


def _sc_agg_body(p0, p1, src0, dst0, src1, dst1, z64, agg_out,
                 srcv, dstv, r0, r1, r2, r3, t0, t1, t2, t3, acc,
                 s0_, s1_, s2_, s3_, u0, u1, u2, u3):
    c = lax.axis_index("c")
    s = lax.axis_index("s")
    pltpu.sync_copy(z64, acc.at[pl.ds(s * ZR, ZR)])

    @pl.when(c == 0)
    def _():
        pltpu.sync_copy(src0.at[pl.ds(s * CPS, CPS)], srcv)
        pltpu.sync_copy(dst0.at[pl.ds(s * CPS, CPS)], dstv)

    @pl.when(c == 1)
    def _():
        pltpu.sync_copy(src1.at[pl.ds(s * CPS, CPS)], srcv)
        pltpu.sync_copy(dst1.at[pl.ds(s * CPS, CPS)], dstv)

    plsc.subcore_barrier()

    bufs = ((r0, s0_), (r1, s1_), (r2, s2_), (r3, s3_))
    stages = ((t0, u0), (t1, u1), (t2, u2), (t3, u3))

    def run(p):
        def process(prev, b):
            buf, gsem = bufs[b]
            stg, ssem = stages[b]
            pltpu.make_async_copy(p.at[srcv.at[prev]], buf, gsem).wait()
            pltpu.make_async_copy(stg, acc.at[dstv.at[prev]], ssem).wait()

            @plsc.parallel_loop(0, CHUNK, unroll=4)
            def _(r):
                for g in range(2):
                    v = buf[r, pl.ds(g * 32, 32)]
                    ev, od = plsc.unpack(v, format=plsc.PackFormat.INTERLEAVED)
                    stg[r, pl.ds(g * 32, 16)] = ev
                    stg[r, pl.ds(g * 32 + 16, 16)] = od

            pltpu.async_copy(stg, acc.at[dstv.at[prev]], ssem)

        for b in range(NBUF):
            buf, gsem = bufs[b]
            stg, ssem = stages[b]
            pltpu.sync_copy(z64.at[pl.ds(0, CHUNK)], stg)
            pltpu.async_copy(stg, acc.at[dstv.at[b]], ssem)
            pltpu.async_copy(p.at[srcv.at[b]], buf, gsem)

        def group(j, carry):
            for b in range(NBUF):
                process(NBUF * (j - 1) + b, b)
                pltpu.async_copy(p.at[srcv.at[NBUF * j + b]],
                                 bufs[b][0], bufs[b][1])
            return carry

        lax.fori_loop(1, CPS // NBUF, group, 0)
        for b in range(NBUF):
            process(CPS - NBUF + b, b)
        for b in range(NBUF):
            stg, ssem = stages[b]
            pltpu.make_async_copy(
                stg, acc.at[dstv.at[CPS - NBUF + b]], ssem).wait()

    @pl.when(c == 0)
    def _():
        run(p0)

    @pl.when(c == 1)
    def _():
        run(p1)

    plsc.subcore_barrier()
    pltpu.sync_copy(acc.at[pl.ds(s * ZR, ZR)], agg_out.at[c, pl.ds(s * ZR, ZR)])


def _dot_nt(a, b):
    return lax.dot_general(a, b, (((1,), (1,)), ((), ())),
                           preferred_element_type=jnp.float32)


def _dot_nn(a, b):
    return lax.dot_general(a, b, (((1,), (0,)), ((), ())),
                           preferred_element_type=jnp.float32)


def _tc_pre_body(x_ref, wsrc_ref, wu_ref, p0_ref, p1_ref):
    x = x_ref[...]
    outs = [p0_ref, p1_ref]
    for m in range(MT):
        wu2 = wu_ref[m, :, H:]
        b_mat = _dot_nn(wu2, wsrc_ref[m])
        outs[m][...] = _dot_nt(x, b_mat).astype(jnp.bfloat16)


def _tc_layer_body(is_last, din,
                   h_ref, agg_ref, cnt_ref,
                   wdst_ref, bdst_ref, bsrc_ref, wu_ref, bu_ref,
                   wa1_ref, ba1_ref, wa2_ref, gamma_ref, beta_ref,
                   *rest):
    if is_last:
        wfc_ref, bfc_ref, out_ref = rest
    else:
        wsrcn_ref, wun_ref, h_out_ref, p0_ref, p1_ref = rest

    h = h_ref[...]
    embs = []
    scores = []
    for m in range(MT):
        wu1 = wu_ref[m, :, :H]
        wu2 = wu_ref[m, :, H:]
        a_mat = _dot_nn(wu1, wdst_ref[m])
        cvec = (bu_ref[m] + _dot_nt(bdst_ref[m], wu1)
                + _dot_nt(bsrc_ref[m], wu2))
        inv = 1.0 / jnp.maximum(cnt_ref[m, :N, 0:1], 1.0)
        emb = _dot_nt(h, a_mat) + agg_ref[m, :N, :] * inv + cvec
        t = jnp.tanh(_dot_nt(emb, wa1_ref[...]) + ba1_ref[...])
        scores.append(jnp.sum(t * wa2_ref[...]) * (1.0 / N))
        embs.append(emb)

    a0 = 1.0 / (1.0 + jnp.exp(scores[1] - scores[0]))
    hc = a0 * embs[0] + (1.0 - a0) * embs[1]

    mu = jnp.sum(hc, axis=0, keepdims=True) * (1.0 / N)
    d = hc - mu
    var = jnp.sum(d * d, axis=0, keepdims=True) * (1.0 / N)
    hn = d * lax.rsqrt(var + 1.0) * gamma_ref[...] + beta_ref[...]
    ho = jnp.where(hn >= 0, hn, 0.01 * hn)

    if is_last:
        out_ref[...] = (jnp.sum(ho * wfc_ref[...], axis=1, keepdims=True)
                        + bfc_ref[...])
    else:
        h_out_ref[...] = ho
        outs = [p0_ref, p1_ref]
        for m in range(MT):
            wu2n = wun_ref[m, :, H:]
            b_next = _dot_nn(wu2n, wsrcn_ref[m])
            outs[m][...] = _dot_nt(ho, b_next).astype(jnp.bfloat16)


_TC_PARAMS = pltpu.CompilerParams(vmem_limit_bytes=100 * 1024 * 1024)

_tc_pre = pl.pallas_call(
    _tc_pre_body,
    out_shape=[jax.ShapeDtypeStruct((N, H), jnp.bfloat16)] * MT,
    compiler_params=_TC_PARAMS,
)


def _make_tc_layer(is_last, din):
    if is_last:
        out_shape = jax.ShapeDtypeStruct((N, 1), jnp.float32)
    else:
        out_shape = [jax.ShapeDtypeStruct((N, H), jnp.float32),
                     jax.ShapeDtypeStruct((N, H), jnp.bfloat16),
                     jax.ShapeDtypeStruct((N, H), jnp.bfloat16)]
    return pl.pallas_call(
        functools.partial(_tc_layer_body, is_last, din),
        out_shape=out_shape,
        compiler_params=_TC_PARAMS,
    )


_tc_layer_first = _make_tc_layer(False, D_IN)
_tc_layer_mid = _make_tc_layer(False, H)
_tc_layer_last = _make_tc_layer(True, H)


def _stack(layer, name, key):
    return jnp.stack([layer["convs"][m][name][key] for m in range(MT)])


def _stack_bias(layer, name):
    return jnp.stack([layer["convs"][m][name]["b"][None, :] for m in range(MT)])


_PERM = np.concatenate(
    [np.concatenate([32 * g + 2 * np.arange(16), 32 * g + 2 * np.arange(16) + 1])
     for g in range(H // 32)])


def kernel(x, edge_index_0, edge_index_1, params):
    pad = E_PAD - E
    srcs, dsts = [], []
    dummy = N + (jnp.arange(pad, dtype=jnp.int32) % (N_PAD - N))
    for ei in (edge_index_0, edge_index_1):
        src = jnp.concatenate([ei[0], jnp.zeros((pad,), jnp.int32)])
        dst = jnp.concatenate([ei[1], dummy])
        srcs.append(src.reshape(E_PAD // CHUNK, CHUNK))
        dsts.append(dst.reshape(E_PAD // CHUNK, CHUNK))

    z16 = jnp.zeros((ZR, 16), jnp.float32)
    ones16 = jnp.ones((CHUNK, 16), jnp.float32)
    z64 = jnp.zeros((ZR, H), jnp.float32)

    cnt = _get_sc_counts()(dsts[0], dsts[1], z16, ones16)

    layers = params["layers"]
    p0, p1 = _tc_pre(x, _stack(layers[0], "lin_src", "W"),
                     _stack(layers[0], "lin_update", "W"))

    h = x
    for i in range(L):
        lp = layers[i]
        agg = _get_sc_agg()(p0, p1, srcs[0], dsts[0], srcs[1], dsts[1], z64)
        wdst = _stack(lp, "lin_dst", "W")
        if i > 0:
            wdst = wdst[:, :, _PERM]
        common = (
            h, agg, cnt,
            wdst, _stack_bias(lp, "lin_dst"),
            _stack_bias(lp, "lin_src"),
            _stack(lp, "lin_update", "W")[:, _PERM, :],
            _stack_bias(lp, "lin_update")[:, :, _PERM],
            lp["attn1"]["W"][:, _PERM], lp["attn1"]["b"][None, :],
            lp["attn2"]["W"],
            lp["bn_gamma"][None, _PERM], lp["bn_beta"][None, _PERM],
        )
        if i == L - 1:
            out = _tc_layer_last(*common, params["fc"]["W"][:, _PERM],
                                 params["fc"]["b"][None, :])
        else:
            nxt = layers[i + 1]
            tc = _tc_layer_first if i == 0 else _tc_layer_mid
            h, p0, p1 = tc(*common, _stack(nxt, "lin_src", "W")[:, :, _PERM],
                           _stack(nxt, "lin_update", "W"))
    return out
